# baseline (speedup 1.0000x reference)
import functools

import jax
import jax.numpy as jnp
from jax import lax
from jax.experimental import pallas as pl
from jax.experimental.pallas import tpu as pltpu
from jax.experimental.pallas import tpu_sc as plsc

_D = 128
_NC = 2
_NS = 16
_NW = _NC * _NS


def _pad_up(v, m):
    return ((v + m - 1) // m) * m


@functools.lru_cache(maxsize=None)
def _make_deg(npad, epad):
    per_w = epad // _NW
    n_chunks = per_w // 128
    rpt = npad // _NS
    mesh = plsc.VectorSubcoreMesh(core_axis_name="c", subcore_axis_name="s")

    def body(dst_hbm, out_hbm, didx_all, ones_v, zb_v, sems, deg_sh):
        c = lax.axis_index("c")
        s = lax.axis_index("s")
        wid = c * _NS + s
        pltpu.sync_copy(dst_hbm.at[pl.ds(wid * n_chunks, n_chunks)], didx_all)
        for l in range(8):
            ones_v[pl.ds(l * 16, 16)] = jnp.full((16,), 1.0, jnp.float32)
        for i in range(rpt // 16):
            zb_v[pl.ds(i * 16, 16)] = jnp.zeros((16,), jnp.float32)
        pltpu.sync_copy(zb_v, deg_sh.at[pl.ds(s * rpt, rpt)])
        plsc.subcore_barrier()

        def group(g, carry):
            base = g * 4
            hs = []
            for b in range(4):
                hs.append(pltpu.async_copy(
                    ones_v, deg_sh.at[didx_all.at[base + b]], sems[b],
                    add=True))
            for b in range(4):
                hs[b].wait()
            return carry

        lax.fori_loop(0, n_chunks // 4, group, 0)
        plsc.subcore_barrier()
        pltpu.sync_copy(deg_sh.at[pl.ds(s * rpt, rpt)],
                        out_hbm.at[c, pl.ds(s * rpt, rpt)])

    return pl.kernel(
        body,
        out_type=jax.ShapeDtypeStruct((_NC, npad), jnp.float32),
        mesh=mesh,
        scratch_types=[
            pltpu.VMEM((n_chunks, 128), jnp.int32),
            pltpu.VMEM((128,), jnp.float32),
            pltpu.VMEM((rpt,), jnp.float32),
            [pltpu.SemaphoreType.DMA] * 4,
            pltpu.VMEM_SHARED((npad,), jnp.float32),
        ],
    )


_CH = 64
_NB = 4
_NH = 4
_FAST = 1
_FRAC_NUM = 7
_FRAC_DEN = 8


@functools.lru_cache(maxsize=None)
def _make_prop(npad, epad):
    per_w = epad // _NW
    n_chunks = per_w // _CH
    hc = n_chunks // _NH
    n_groups = hc // _NB
    rpt = npad // _NS
    mesh = plsc.VectorSubcoreMesh(core_axis_name="c", subcore_axis_name="s")

    n_blocks = _NW * _NH
    fast_blocks = (_NH * _NW * _FRAC_NUM) // (_FRAC_DEN * _NS)
    slow_blocks = n_blocks // _NS - fast_blocks
    max_blocks = max(fast_blocks, slow_blocks)

    def body(src_hbm, dst_hbm, y_hbm, z_hbm, out_hbm,
             sidx_all, didx_all, rows, gsems, ssems, acc_sh):
        c = lax.axis_index("c")
        s = lax.axis_index("s")
        pltpu.sync_copy(z_hbm.at[pl.ds(s * rpt, rpt)],
                        acc_sh.at[pl.ds(s * rpt, rpt)])
        plsc.subcore_barrier()

        is_fast = (c == _FAST)
        my_blocks = jnp.where(is_fast, fast_blocks, slow_blocks)
        blk0 = jnp.where(is_fast, s * fast_blocks,
                         _NS * fast_blocks + s * slow_blocks)

        for k in range(max_blocks):
            @pl.when(k < my_blocks)
            def _do_block():
                off = (blk0 + k) * hc
                pltpu.sync_copy(src_hbm.at[pl.ds(off, hc)], sidx_all)
                pltpu.sync_copy(dst_hbm.at[pl.ds(off, hc)], didx_all)

                def group(g, carry):
                    base = g * _NB
                    gh = []
                    for b in range(_NB):
                        @pl.when(g > 0)
                        def _drain():
                            pltpu.make_async_copy(
                                rows.at[b],
                                acc_sh.at[didx_all.at[base + b]],
                                ssems[b]).wait()
                        gh.append(pltpu.async_copy(
                            y_hbm.at[sidx_all.at[base + b]], rows.at[b],
                            gsems[b]))
                    for b in range(_NB):
                        gh[b].wait()
                        pltpu.async_copy(
                            rows.at[b], acc_sh.at[didx_all.at[base + b]],
                            ssems[b], add=True)
                    return carry

                lax.fori_loop(0, n_groups, group, 0)
                for b in range(_NB):
                    pltpu.make_async_copy(
                        rows.at[b], acc_sh.at[didx_all.at[b]],
                        ssems[b]).wait()

        plsc.subcore_barrier()
        pltpu.sync_copy(acc_sh.at[pl.ds(s * rpt, rpt)],
                        out_hbm.at[c, pl.ds(s * rpt, rpt)])

    return pl.kernel(
        body,
        out_type=jax.ShapeDtypeStruct((_NC, npad, _D), jnp.float32),
        mesh=mesh,
        scratch_types=[
            pltpu.VMEM((hc, _CH), jnp.int32),
            pltpu.VMEM((hc, _CH), jnp.int32),
            pltpu.VMEM((_NB, _CH, _D), jnp.float32),
            [pltpu.SemaphoreType.DMA] * _NB,
            [pltpu.SemaphoreType.DMA] * _NB,
            pltpu.VMEM_SHARED((npad, _D), jnp.float32),
        ],
    )


def _scale_body(d0_ref, d1_ref, x_ref, y_ref, dis_ref):
    deg = d0_ref[...] + d1_ref[...] + 1.0
    dis = lax.rsqrt(deg)
    dis_ref[...] = dis
    y_ref[...] = x_ref[...] * dis


@functools.lru_cache(maxsize=None)
def _make_scale(npad, blk):
    grid = npad // blk
    return pl.pallas_call(
        _scale_body,
        grid=(grid,),
        in_specs=[
            pl.BlockSpec((blk, 1), lambda i: (i, 0)),
            pl.BlockSpec((blk, 1), lambda i: (i, 0)),
            pl.BlockSpec((blk, _D), lambda i: (i, 0)),
        ],
        out_specs=[
            pl.BlockSpec((blk, _D), lambda i: (i, 0)),
            pl.BlockSpec((blk, 1), lambda i: (i, 0)),
        ],
        out_shape=[
            jax.ShapeDtypeStruct((npad, _D), jnp.float32),
            jax.ShapeDtypeStruct((npad, 1), jnp.float32),
        ],
    )


def _final_body(s0_ref, s1_ref, y_ref, dis_ref, w_ref, b_ref, out_ref):
    z = (s0_ref[...] + s1_ref[...] + y_ref[...]) * dis_ref[...]
    o = jnp.dot(z, w_ref[...], preferred_element_type=jnp.float32) + b_ref[...]
    m = jnp.max(o, axis=1, keepdims=True)
    ex = jnp.exp(o - m)
    lse = jnp.log(jnp.sum(ex, axis=1, keepdims=True))
    out_ref[...] = o - m - lse


@functools.lru_cache(maxsize=None)
def _make_final(npad, blk):
    grid = npad // blk
    return pl.pallas_call(
        _final_body,
        grid=(grid,),
        in_specs=[
            pl.BlockSpec((blk, _D), lambda i: (i, 0)),
            pl.BlockSpec((blk, _D), lambda i: (i, 0)),
            pl.BlockSpec((blk, _D), lambda i: (i, 0)),
            pl.BlockSpec((blk, 1), lambda i: (i, 0)),
            pl.BlockSpec((_D, _D), lambda i: (0, 0)),
            pl.BlockSpec((1, _D), lambda i: (0, 0)),
        ],
        out_specs=pl.BlockSpec((blk, _D), lambda i: (i, 0)),
        out_shape=jax.ShapeDtypeStruct((npad, _D), jnp.float32),
    )


@jax.jit
def kernel(x, edge_index, W, b):
    n, d = x.shape
    e = edge_index.shape[1]
    npad = _pad_up(n, 1024)
    per_w = _pad_up(-(-e // _NW), 1024)
    epad = per_w * _NW

    fill = jnp.full((epad - e,), npad - 1, jnp.int32)
    src1 = jnp.concatenate([edge_index[0], fill])
    dst1 = jnp.concatenate([edge_index[1], fill])
    src = src1.reshape(epad // _CH, _CH)
    dst = dst1.reshape(epad // _CH, _CH)
    dsth = dst1.reshape(epad // 128, 128)
    xp = jnp.pad(x, ((0, npad - n), (0, 0)))
    zeros2 = jnp.zeros((npad, d), jnp.float32)

    degp = _make_deg(npad, epad)(dsth)
    y, dis = _make_scale(npad, 2048)(
        degp[0].reshape(npad, 1), degp[1].reshape(npad, 1), xp)
    sp = _make_prop(npad, epad)(src, dst, y, zeros2)
    out = _make_final(npad, 1024)(sp[0], sp[1], y, dis, W, b.reshape(1, d))
    return out[:n]


# speedup vs baseline: 1.0219x; 1.0219x over previous
import functools

import jax
import jax.numpy as jnp
from jax import lax
from jax.experimental import pallas as pl
from jax.experimental.pallas import tpu as pltpu
from jax.experimental.pallas import tpu_sc as plsc

_D = 128
_NC = 2
_NS = 16
_NW = _NC * _NS


def _pad_up(v, m):
    return ((v + m - 1) // m) * m


@functools.lru_cache(maxsize=None)
def _make_deg(npad, epad):
    per_w = epad // _NW
    n_chunks = per_w // 128
    rpt = npad // _NS
    mesh = plsc.VectorSubcoreMesh(core_axis_name="c", subcore_axis_name="s")

    def body(dst_hbm, out_hbm, didx_all, ones_v, zb_v, sems, deg_sh):
        c = lax.axis_index("c")
        s = lax.axis_index("s")
        wid = c * _NS + s
        pltpu.sync_copy(dst_hbm.at[pl.ds(wid * n_chunks, n_chunks)], didx_all)
        for l in range(8):
            ones_v[pl.ds(l * 16, 16)] = jnp.full((16,), 1.0, jnp.float32)
        for i in range(rpt // 16):
            zb_v[pl.ds(i * 16, 16)] = jnp.zeros((16,), jnp.float32)
        pltpu.sync_copy(zb_v, deg_sh.at[pl.ds(s * rpt, rpt)])
        plsc.subcore_barrier()

        def group(g, carry):
            base = g * 4
            hs = []
            for b in range(4):
                hs.append(pltpu.async_copy(
                    ones_v, deg_sh.at[didx_all.at[base + b]], sems[b],
                    add=True))
            for b in range(4):
                hs[b].wait()
            return carry

        lax.fori_loop(0, n_chunks // 4, group, 0)
        plsc.subcore_barrier()
        pltpu.sync_copy(deg_sh.at[pl.ds(s * rpt, rpt)],
                        out_hbm.at[c, pl.ds(s * rpt, rpt)])

    return pl.kernel(
        body,
        out_type=jax.ShapeDtypeStruct((_NC, npad), jnp.float32),
        mesh=mesh,
        scratch_types=[
            pltpu.VMEM((n_chunks, 128), jnp.int32),
            pltpu.VMEM((128,), jnp.float32),
            pltpu.VMEM((rpt,), jnp.float32),
            [pltpu.SemaphoreType.DMA] * 4,
            pltpu.VMEM_SHARED((npad,), jnp.float32),
        ],
    )


_CH = 64
_NB = 4
_NH = 4
_FAST = 0
_FRAC_NUM = 7
_FRAC_DEN = 8


@functools.lru_cache(maxsize=None)
def _make_prop(npad, epad):
    per_w = epad // _NW
    n_chunks = per_w // _CH
    hc = n_chunks // _NH
    n_groups = hc // _NB
    rpt = npad // _NS
    mesh = plsc.VectorSubcoreMesh(core_axis_name="c", subcore_axis_name="s")

    n_blocks = _NW * _NH
    fast_blocks = (_NH * _NW * _FRAC_NUM) // (_FRAC_DEN * _NS)
    slow_blocks = n_blocks // _NS - fast_blocks
    max_blocks = max(fast_blocks, slow_blocks)

    def body(src_hbm, dst_hbm, y_hbm, z_hbm, out_hbm,
             sidx_all, didx_all, rows, gsems, ssems, acc_sh):
        c = lax.axis_index("c")
        s = lax.axis_index("s")
        pltpu.sync_copy(z_hbm.at[pl.ds(s * rpt, rpt)],
                        acc_sh.at[pl.ds(s * rpt, rpt)])
        plsc.subcore_barrier()

        is_fast = (c == _FAST)
        my_blocks = jnp.where(is_fast, fast_blocks, slow_blocks)
        blk0 = jnp.where(is_fast, s * fast_blocks,
                         _NS * fast_blocks + s * slow_blocks)

        for k in range(max_blocks):
            @pl.when(k < my_blocks)
            def _do_block():
                off = (blk0 + k) * hc
                pltpu.sync_copy(src_hbm.at[pl.ds(off, hc)], sidx_all)
                pltpu.sync_copy(dst_hbm.at[pl.ds(off, hc)], didx_all)

                def group(g, carry):
                    base = g * _NB
                    gh = []
                    for b in range(_NB):
                        @pl.when(g > 0)
                        def _drain():
                            pltpu.make_async_copy(
                                rows.at[b],
                                acc_sh.at[didx_all.at[base + b]],
                                ssems[b]).wait()
                        gh.append(pltpu.async_copy(
                            y_hbm.at[sidx_all.at[base + b]], rows.at[b],
                            gsems[b]))
                    for b in range(_NB):
                        gh[b].wait()
                        pltpu.async_copy(
                            rows.at[b], acc_sh.at[didx_all.at[base + b]],
                            ssems[b], add=True)
                    return carry

                lax.fori_loop(0, n_groups, group, 0)
                for b in range(_NB):
                    pltpu.make_async_copy(
                        rows.at[b], acc_sh.at[didx_all.at[b]],
                        ssems[b]).wait()

        plsc.subcore_barrier()
        pltpu.sync_copy(acc_sh.at[pl.ds(s * rpt, rpt)],
                        out_hbm.at[c, pl.ds(s * rpt, rpt)])

    return pl.kernel(
        body,
        out_type=jax.ShapeDtypeStruct((_NC, npad, _D), jnp.float32),
        mesh=mesh,
        scratch_types=[
            pltpu.VMEM((hc, _CH), jnp.int32),
            pltpu.VMEM((hc, _CH), jnp.int32),
            pltpu.VMEM((_NB, _CH, _D), jnp.float32),
            [pltpu.SemaphoreType.DMA] * _NB,
            [pltpu.SemaphoreType.DMA] * _NB,
            pltpu.VMEM_SHARED((npad, _D), jnp.float32),
        ],
    )


def _scale_body(d0_ref, d1_ref, x_ref, y_ref, dis_ref):
    deg = d0_ref[...] + d1_ref[...] + 1.0
    dis = lax.rsqrt(deg)
    dis_ref[...] = dis
    y_ref[...] = x_ref[...] * dis


@functools.lru_cache(maxsize=None)
def _make_scale(npad, blk):
    grid = npad // blk
    return pl.pallas_call(
        _scale_body,
        grid=(grid,),
        in_specs=[
            pl.BlockSpec((blk, 1), lambda i: (i, 0)),
            pl.BlockSpec((blk, 1), lambda i: (i, 0)),
            pl.BlockSpec((blk, _D), lambda i: (i, 0)),
        ],
        out_specs=[
            pl.BlockSpec((blk, _D), lambda i: (i, 0)),
            pl.BlockSpec((blk, 1), lambda i: (i, 0)),
        ],
        out_shape=[
            jax.ShapeDtypeStruct((npad, _D), jnp.float32),
            jax.ShapeDtypeStruct((npad, 1), jnp.float32),
        ],
    )


def _final_body(s0_ref, s1_ref, y_ref, dis_ref, w_ref, b_ref, out_ref):
    z = (s0_ref[...] + s1_ref[...] + y_ref[...]) * dis_ref[...]
    o = jnp.dot(z, w_ref[...], preferred_element_type=jnp.float32) + b_ref[...]
    m = jnp.max(o, axis=1, keepdims=True)
    ex = jnp.exp(o - m)
    lse = jnp.log(jnp.sum(ex, axis=1, keepdims=True))
    out_ref[...] = o - m - lse


@functools.lru_cache(maxsize=None)
def _make_final(npad, blk):
    grid = npad // blk
    return pl.pallas_call(
        _final_body,
        grid=(grid,),
        in_specs=[
            pl.BlockSpec((blk, _D), lambda i: (i, 0)),
            pl.BlockSpec((blk, _D), lambda i: (i, 0)),
            pl.BlockSpec((blk, _D), lambda i: (i, 0)),
            pl.BlockSpec((blk, 1), lambda i: (i, 0)),
            pl.BlockSpec((_D, _D), lambda i: (0, 0)),
            pl.BlockSpec((1, _D), lambda i: (0, 0)),
        ],
        out_specs=pl.BlockSpec((blk, _D), lambda i: (i, 0)),
        out_shape=jax.ShapeDtypeStruct((npad, _D), jnp.float32),
    )


@jax.jit
def kernel(x, edge_index, W, b):
    n, d = x.shape
    e = edge_index.shape[1]
    npad = _pad_up(n, 1024)
    per_w = _pad_up(-(-e // _NW), 1024)
    epad = per_w * _NW

    fill = jnp.full((epad - e,), npad - 1, jnp.int32)
    src1 = jnp.concatenate([edge_index[0], fill])
    dst1 = jnp.concatenate([edge_index[1], fill])
    src = src1.reshape(epad // _CH, _CH)
    dst = dst1.reshape(epad // _CH, _CH)
    dsth = dst1.reshape(epad // 128, 128)
    xp = jnp.pad(x, ((0, npad - n), (0, 0)))
    zeros2 = jnp.zeros((npad, d), jnp.float32)

    degp = _make_deg(npad, epad)(dsth)
    y, dis = _make_scale(npad, 2048)(
        degp[0].reshape(npad, 1), degp[1].reshape(npad, 1), xp)
    sp = _make_prop(npad, epad)(src, dst, y, zeros2)
    out = _make_final(npad, 1024)(sp[0], sp[1], y, dis, W, b.reshape(1, d))
    return out[:n]


# speedup vs baseline: 1.0231x; 1.0012x over previous
import functools

import jax
import jax.numpy as jnp
from jax import lax
from jax.experimental import pallas as pl
from jax.experimental.pallas import tpu as pltpu
from jax.experimental.pallas import tpu_sc as plsc

_D = 128
_NC = 2
_NS = 16
_NW = _NC * _NS


def _pad_up(v, m):
    return ((v + m - 1) // m) * m


@functools.lru_cache(maxsize=None)
def _make_deg(npad, epad):
    per_w = epad // _NW
    n_chunks = per_w // 128
    rpt = npad // _NS
    mesh = plsc.VectorSubcoreMesh(core_axis_name="c", subcore_axis_name="s")

    def body(dst_hbm, out_hbm, didx_all, ones_v, zb_v, sems, deg_sh):
        c = lax.axis_index("c")
        s = lax.axis_index("s")
        wid = c * _NS + s
        pltpu.sync_copy(dst_hbm.at[pl.ds(wid * n_chunks, n_chunks)], didx_all)
        for l in range(8):
            ones_v[pl.ds(l * 16, 16)] = jnp.full((16,), 1.0, jnp.float32)
        for i in range(rpt // 16):
            zb_v[pl.ds(i * 16, 16)] = jnp.zeros((16,), jnp.float32)
        pltpu.sync_copy(zb_v, deg_sh.at[pl.ds(s * rpt, rpt)])
        plsc.subcore_barrier()

        def group(g, carry):
            base = g * 4
            hs = []
            for b in range(4):
                hs.append(pltpu.async_copy(
                    ones_v, deg_sh.at[didx_all.at[base + b]], sems[b],
                    add=True))
            for b in range(4):
                hs[b].wait()
            return carry

        lax.fori_loop(0, n_chunks // 4, group, 0)
        plsc.subcore_barrier()
        pltpu.sync_copy(deg_sh.at[pl.ds(s * rpt, rpt)],
                        out_hbm.at[c, pl.ds(s * rpt, rpt)])

    return pl.kernel(
        body,
        out_type=jax.ShapeDtypeStruct((_NC, npad), jnp.float32),
        mesh=mesh,
        scratch_types=[
            pltpu.VMEM((n_chunks, 128), jnp.int32),
            pltpu.VMEM((128,), jnp.float32),
            pltpu.VMEM((rpt,), jnp.float32),
            [pltpu.SemaphoreType.DMA] * 4,
            pltpu.VMEM_SHARED((npad,), jnp.float32),
        ],
    )


_CH = 64
_NB = 4
_NH = 4
_FAST = 0
_FRAC_NUM = 7
_FRAC_DEN = 8


@functools.lru_cache(maxsize=None)
def _make_prop(npad, epad):
    per_w = epad // _NW
    n_chunks = per_w // _CH
    hc = n_chunks // _NH
    n_groups = hc // _NB
    rpt = npad // _NS
    mesh = plsc.VectorSubcoreMesh(core_axis_name="c", subcore_axis_name="s")

    n_blocks = _NW * _NH
    fast_blocks = (_NH * _NW * _FRAC_NUM) // (_FRAC_DEN * _NS)
    slow_blocks = n_blocks // _NS - fast_blocks
    max_blocks = max(fast_blocks, slow_blocks)

    def body(src_hbm, dst_hbm, y_hbm, z_hbm, out_hbm,
             sidx_all, didx_all, rows, gsems, ssems, isems, acc_sh):
        c = lax.axis_index("c")
        s = lax.axis_index("s")
        pltpu.sync_copy(z_hbm.at[pl.ds(s * rpt, rpt)],
                        acc_sh.at[pl.ds(s * rpt, rpt)])
        plsc.subcore_barrier()

        is_fast = (c == _FAST)
        my_blocks = jnp.where(is_fast, fast_blocks, slow_blocks)
        blk0 = jnp.where(is_fast, s * fast_blocks,
                         _NS * fast_blocks + s * slow_blocks)

        for k in range(max_blocks):
            @pl.when(k < my_blocks)
            def _do_block():
                off = (blk0 + k) * hc
                ih0 = pltpu.async_copy(src_hbm.at[pl.ds(off, hc)], sidx_all,
                                       isems[0])
                ih1 = pltpu.async_copy(dst_hbm.at[pl.ds(off, hc)], didx_all,
                                       isems[1])
                ih0.wait()
                ih1.wait()

                def group(g, carry):
                    base = g * _NB
                    gh = []
                    for b in range(_NB):
                        @pl.when(g > 0)
                        def _drain():
                            pltpu.make_async_copy(
                                rows.at[b],
                                acc_sh.at[didx_all.at[base + b]],
                                ssems[b]).wait()
                        gh.append(pltpu.async_copy(
                            y_hbm.at[sidx_all.at[base + b]], rows.at[b],
                            gsems[b]))
                    for b in range(_NB):
                        gh[b].wait()
                        pltpu.async_copy(
                            rows.at[b], acc_sh.at[didx_all.at[base + b]],
                            ssems[b], add=True)
                    return carry

                lax.fori_loop(0, n_groups, group, 0)
                for b in range(_NB):
                    pltpu.make_async_copy(
                        rows.at[b], acc_sh.at[didx_all.at[b]],
                        ssems[b]).wait()

        plsc.subcore_barrier()
        pltpu.sync_copy(acc_sh.at[pl.ds(s * rpt, rpt)],
                        out_hbm.at[c, pl.ds(s * rpt, rpt)])

    return pl.kernel(
        body,
        out_type=jax.ShapeDtypeStruct((_NC, npad, _D), jnp.float32),
        mesh=mesh,
        scratch_types=[
            pltpu.VMEM((hc, _CH), jnp.int32),
            pltpu.VMEM((hc, _CH), jnp.int32),
            pltpu.VMEM((_NB, _CH, _D), jnp.float32),
            [pltpu.SemaphoreType.DMA] * _NB,
            [pltpu.SemaphoreType.DMA] * _NB,
            [pltpu.SemaphoreType.DMA] * 2,
            pltpu.VMEM_SHARED((npad, _D), jnp.float32),
        ],
    )


def _scale_body(d0_ref, d1_ref, x_ref, y_ref, dis_ref):
    deg = d0_ref[...] + d1_ref[...] + 1.0
    dis = lax.rsqrt(deg)
    dis_ref[...] = dis
    y_ref[...] = x_ref[...] * dis


@functools.lru_cache(maxsize=None)
def _make_scale(npad, blk):
    grid = npad // blk
    return pl.pallas_call(
        _scale_body,
        grid=(grid,),
        in_specs=[
            pl.BlockSpec((blk, 1), lambda i: (i, 0)),
            pl.BlockSpec((blk, 1), lambda i: (i, 0)),
            pl.BlockSpec((blk, _D), lambda i: (i, 0)),
        ],
        out_specs=[
            pl.BlockSpec((blk, _D), lambda i: (i, 0)),
            pl.BlockSpec((blk, 1), lambda i: (i, 0)),
        ],
        out_shape=[
            jax.ShapeDtypeStruct((npad, _D), jnp.float32),
            jax.ShapeDtypeStruct((npad, 1), jnp.float32),
        ],
    )


def _final_body(s0_ref, s1_ref, y_ref, dis_ref, w_ref, b_ref, out_ref):
    z = (s0_ref[...] + s1_ref[...] + y_ref[...]) * dis_ref[...]
    o = jnp.dot(z, w_ref[...], preferred_element_type=jnp.float32) + b_ref[...]
    m = jnp.max(o, axis=1, keepdims=True)
    ex = jnp.exp(o - m)
    lse = jnp.log(jnp.sum(ex, axis=1, keepdims=True))
    out_ref[...] = o - m - lse


@functools.lru_cache(maxsize=None)
def _make_final(npad, blk):
    grid = npad // blk
    return pl.pallas_call(
        _final_body,
        grid=(grid,),
        in_specs=[
            pl.BlockSpec((blk, _D), lambda i: (i, 0)),
            pl.BlockSpec((blk, _D), lambda i: (i, 0)),
            pl.BlockSpec((blk, _D), lambda i: (i, 0)),
            pl.BlockSpec((blk, 1), lambda i: (i, 0)),
            pl.BlockSpec((_D, _D), lambda i: (0, 0)),
            pl.BlockSpec((1, _D), lambda i: (0, 0)),
        ],
        out_specs=pl.BlockSpec((blk, _D), lambda i: (i, 0)),
        out_shape=jax.ShapeDtypeStruct((npad, _D), jnp.float32),
    )


@jax.jit
def kernel(x, edge_index, W, b):
    n, d = x.shape
    e = edge_index.shape[1]
    npad = _pad_up(n, 1024)
    per_w = _pad_up(-(-e // _NW), 1024)
    epad = per_w * _NW

    fill = jnp.full((epad - e,), npad - 1, jnp.int32)
    src1 = jnp.concatenate([edge_index[0], fill])
    dst1 = jnp.concatenate([edge_index[1], fill])
    src = src1.reshape(epad // _CH, _CH)
    dst = dst1.reshape(epad // _CH, _CH)
    dsth = dst1.reshape(epad // 128, 128)
    xp = jnp.pad(x, ((0, npad - n), (0, 0)))
    zeros2 = jnp.zeros((npad, d), jnp.float32)

    degp = _make_deg(npad, epad)(dsth)
    y, dis = _make_scale(npad, 2048)(
        degp[0].reshape(npad, 1), degp[1].reshape(npad, 1), xp)
    sp = _make_prop(npad, epad)(src, dst, y, zeros2)
    out = _make_final(npad, 1024)(sp[0], sp[1], y, dis, W, b.reshape(1, d))
    return out[:n]


# speedup vs baseline: 1.0501x; 1.0265x over previous
import functools

import jax
import jax.numpy as jnp
from jax import lax
from jax.experimental import pallas as pl
from jax.experimental.pallas import tpu as pltpu
from jax.experimental.pallas import tpu_sc as plsc

_D = 128
_NC = 2
_NS = 16
_NW = _NC * _NS


def _pad_up(v, m):
    return ((v + m - 1) // m) * m


@functools.lru_cache(maxsize=None)
def _make_deg(npad, epad):
    per_w = epad // _NW
    n_chunks = per_w // 128
    rpt = npad // _NS
    mesh = plsc.VectorSubcoreMesh(core_axis_name="c", subcore_axis_name="s")

    def body(dst_hbm, out_hbm, didx_all, ones_v, zb_v, sems, deg_sh):
        c = lax.axis_index("c")
        s = lax.axis_index("s")
        wid = c * _NS + s
        pltpu.sync_copy(dst_hbm.at[pl.ds(wid * n_chunks, n_chunks)], didx_all)
        for l in range(8):
            ones_v[pl.ds(l * 16, 16)] = jnp.full((16,), 1.0, jnp.float32)
        for i in range(rpt // 16):
            zb_v[pl.ds(i * 16, 16)] = jnp.zeros((16,), jnp.float32)
        pltpu.sync_copy(zb_v, deg_sh.at[pl.ds(s * rpt, rpt)])
        plsc.subcore_barrier()

        def group(g, carry):
            base = g * 4
            hs = []
            for b in range(4):
                hs.append(pltpu.async_copy(
                    ones_v, deg_sh.at[didx_all.at[base + b]], sems[b],
                    add=True))
            for b in range(4):
                hs[b].wait()
            return carry

        lax.fori_loop(0, n_chunks // 4, group, 0)
        plsc.subcore_barrier()
        pltpu.sync_copy(deg_sh.at[pl.ds(s * rpt, rpt)],
                        out_hbm.at[c, pl.ds(s * rpt, rpt)])

    return pl.kernel(
        body,
        out_type=jax.ShapeDtypeStruct((_NC, npad), jnp.float32),
        mesh=mesh,
        scratch_types=[
            pltpu.VMEM((n_chunks, 128), jnp.int32),
            pltpu.VMEM((128,), jnp.float32),
            pltpu.VMEM((rpt,), jnp.float32),
            [pltpu.SemaphoreType.DMA] * 4,
            pltpu.VMEM_SHARED((npad,), jnp.float32),
        ],
    )


_CH = 64
_NB = 4
_NH = 5
_FAST = 0
_FRAC_NUM = 9
_FRAC_DEN = 10


@functools.lru_cache(maxsize=None)
def _make_prop(npad, epad):
    per_w = epad // _NW
    n_chunks = per_w // _CH
    hc = n_chunks // _NH
    n_groups = hc // _NB
    rpt = npad // _NS
    mesh = plsc.VectorSubcoreMesh(core_axis_name="c", subcore_axis_name="s")

    n_blocks = _NW * _NH
    fast_blocks = (_NH * _NW * _FRAC_NUM) // (_FRAC_DEN * _NS)
    slow_blocks = n_blocks // _NS - fast_blocks
    max_blocks = max(fast_blocks, slow_blocks)

    def body(src_hbm, dst_hbm, y_hbm, z_hbm, out_hbm,
             sidx_all, didx_all, rows, gsems, ssems, isems, acc_sh):
        c = lax.axis_index("c")
        s = lax.axis_index("s")
        pltpu.sync_copy(z_hbm.at[pl.ds(s * rpt, rpt)],
                        acc_sh.at[pl.ds(s * rpt, rpt)])
        plsc.subcore_barrier()

        is_fast = (c == _FAST)
        my_blocks = jnp.where(is_fast, fast_blocks, slow_blocks)
        blk0 = jnp.where(is_fast, s * fast_blocks,
                         _NS * fast_blocks + s * slow_blocks)

        for k in range(max_blocks):
            @pl.when(k < my_blocks)
            def _do_block():
                off = (blk0 + k) * hc
                ih0 = pltpu.async_copy(src_hbm.at[pl.ds(off, hc)], sidx_all,
                                       isems[0])
                ih1 = pltpu.async_copy(dst_hbm.at[pl.ds(off, hc)], didx_all,
                                       isems[1])
                ih0.wait()
                ih1.wait()

                def group(g, carry):
                    base = g * _NB
                    gh = []
                    for b in range(_NB):
                        @pl.when(g > 0)
                        def _drain():
                            pltpu.make_async_copy(
                                rows.at[b],
                                acc_sh.at[didx_all.at[base + b]],
                                ssems[b]).wait()
                        gh.append(pltpu.async_copy(
                            y_hbm.at[sidx_all.at[base + b]], rows.at[b],
                            gsems[b]))
                    for b in range(_NB):
                        gh[b].wait()
                        pltpu.async_copy(
                            rows.at[b], acc_sh.at[didx_all.at[base + b]],
                            ssems[b], add=True)
                    return carry

                lax.fori_loop(0, n_groups, group, 0)
                for b in range(_NB):
                    pltpu.make_async_copy(
                        rows.at[b], acc_sh.at[didx_all.at[b]],
                        ssems[b]).wait()

        plsc.subcore_barrier()
        pltpu.sync_copy(acc_sh.at[pl.ds(s * rpt, rpt)],
                        out_hbm.at[c, pl.ds(s * rpt, rpt)])

    return pl.kernel(
        body,
        out_type=jax.ShapeDtypeStruct((_NC, npad, _D), jnp.float32),
        mesh=mesh,
        scratch_types=[
            pltpu.VMEM((hc, _CH), jnp.int32),
            pltpu.VMEM((hc, _CH), jnp.int32),
            pltpu.VMEM((_NB, _CH, _D), jnp.float32),
            [pltpu.SemaphoreType.DMA] * _NB,
            [pltpu.SemaphoreType.DMA] * _NB,
            [pltpu.SemaphoreType.DMA] * 2,
            pltpu.VMEM_SHARED((npad, _D), jnp.float32),
        ],
    )


def _scale_body(d0_ref, d1_ref, x_ref, y_ref, dis_ref):
    deg = d0_ref[...] + d1_ref[...] + 1.0
    dis = lax.rsqrt(deg)
    dis_ref[...] = dis
    y_ref[...] = x_ref[...] * dis


@functools.lru_cache(maxsize=None)
def _make_scale(npad, blk):
    grid = npad // blk
    return pl.pallas_call(
        _scale_body,
        grid=(grid,),
        in_specs=[
            pl.BlockSpec((blk, 1), lambda i: (i, 0)),
            pl.BlockSpec((blk, 1), lambda i: (i, 0)),
            pl.BlockSpec((blk, _D), lambda i: (i, 0)),
        ],
        out_specs=[
            pl.BlockSpec((blk, _D), lambda i: (i, 0)),
            pl.BlockSpec((blk, 1), lambda i: (i, 0)),
        ],
        out_shape=[
            jax.ShapeDtypeStruct((npad, _D), jnp.float32),
            jax.ShapeDtypeStruct((npad, 1), jnp.float32),
        ],
    )


def _final_body(s0_ref, s1_ref, y_ref, dis_ref, w_ref, b_ref, out_ref):
    z = (s0_ref[...] + s1_ref[...] + y_ref[...]) * dis_ref[...]
    o = jnp.dot(z, w_ref[...], preferred_element_type=jnp.float32) + b_ref[...]
    m = jnp.max(o, axis=1, keepdims=True)
    ex = jnp.exp(o - m)
    lse = jnp.log(jnp.sum(ex, axis=1, keepdims=True))
    out_ref[...] = o - m - lse


@functools.lru_cache(maxsize=None)
def _make_final(npad, blk):
    grid = npad // blk
    return pl.pallas_call(
        _final_body,
        grid=(grid,),
        in_specs=[
            pl.BlockSpec((blk, _D), lambda i: (i, 0)),
            pl.BlockSpec((blk, _D), lambda i: (i, 0)),
            pl.BlockSpec((blk, _D), lambda i: (i, 0)),
            pl.BlockSpec((blk, 1), lambda i: (i, 0)),
            pl.BlockSpec((_D, _D), lambda i: (0, 0)),
            pl.BlockSpec((1, _D), lambda i: (0, 0)),
        ],
        out_specs=pl.BlockSpec((blk, _D), lambda i: (i, 0)),
        out_shape=jax.ShapeDtypeStruct((npad, _D), jnp.float32),
    )


@jax.jit
def kernel(x, edge_index, W, b):
    n, d = x.shape
    e = edge_index.shape[1]
    npad = _pad_up(n, 1024)
    per_w = _pad_up(-(-e // _NW), 1024)
    epad = per_w * _NW

    fill = jnp.full((epad - e,), npad - 1, jnp.int32)
    src1 = jnp.concatenate([edge_index[0], fill])
    dst1 = jnp.concatenate([edge_index[1], fill])
    src = src1.reshape(epad // _CH, _CH)
    dst = dst1.reshape(epad // _CH, _CH)
    dsth = dst1.reshape(epad // 128, 128)
    xp = jnp.pad(x, ((0, npad - n), (0, 0)))
    zeros2 = jnp.zeros((npad, d), jnp.float32)

    degp = _make_deg(npad, epad)(dsth)
    y, dis = _make_scale(npad, 2048)(
        degp[0].reshape(npad, 1), degp[1].reshape(npad, 1), xp)
    sp = _make_prop(npad, epad)(src, dst, y, zeros2)
    out = _make_final(npad, 1024)(sp[0], sp[1], y, dis, W, b.reshape(1, d))
    return out[:n]
